# baseline (speedup 1.0000x reference)
import functools

import jax
import jax.numpy as jnp
from jax import lax
from jax.experimental import pallas as pl
from jax.experimental.pallas import tpu as pltpu
from jax.experimental.pallas import tpu_sc as plsc

IN_F = 4096
OUT_F = 4096
NNZ = 167772
B = 1024

N_W = IN_F * OUT_F
NC, NS = 2, 16
G = 84
PW = G * 128
NNZP = NS * PW
NCHUNK = 16
CH = N_W // NCHUNK
SL = CH // NS
ZB = 16384

_mesh = plsc.VectorSubcoreMesh(
    core_axis_name="c", subcore_axis_name="s", num_cores=NC, num_subcores=NS
)


G1 = G + 1

_DENSIFY_SCRATCH = [
    pltpu.VMEM((PW,), jnp.int32),
    pltpu.VMEM((G, 128), jnp.float32),
    pltpu.VMEM((G, 128), jnp.int32),
    pltpu.VMEM((ZB,), jnp.float32),
    pltpu.VMEM_SHARED((CH + 2048,), jnp.float32),
    pltpu.SemaphoreType.DMA,
    pltpu.SemaphoreType.DMA,
]


def _densify_body(fidx_hbm, val_hbm, wd_hbm, fidx_v, val_v, li_v, zbuf, spmem,
                  sem, zsem):
    cid = lax.axis_index("c")
    sid = lax.axis_index("s")

    @pl.loop(0, ZB // 16)
    def _zero(i):
        zbuf[pl.ds(i * 16, 16)] = jnp.zeros((16,), jnp.float32)

    pltpu.sync_copy(fidx_hbm.at[sid], fidx_v)
    pltpu.sync_copy(val_hbm.at[sid], val_v)

    for k in range(NCHUNK // NC):
        base = (k * NC + cid) * CH

        for z in range(SL // ZB):
            pltpu.async_copy(zbuf, spmem.at[pl.ds(sid * SL + z * ZB, ZB)], zsem)

        @pl.loop(0, G, unroll=4)
        def _locals(r):
            for cc in range(128 // 16):
                v = fidx_v[pl.ds(r * 128 + cc * 16, 16)]
                li = v - base
                ok = (li >= 0) & (li < CH)
                li_v[r, pl.ds(cc * 16, 16)] = jnp.where(ok, li, CH + (v & 2047))

        for z in range(SL // ZB):
            pltpu.make_async_copy(
                zbuf, spmem.at[pl.ds(sid * SL + z * ZB, ZB)], zsem
            ).wait()
        plsc.subcore_barrier()

        @pl.loop(0, G)
        def _scatter(r):
            pltpu.async_copy(val_v.at[r], spmem.at[li_v.at[r]], sem, add=True)

        @pl.loop(0, G)
        def _drain(r):
            pltpu.make_async_copy(val_v.at[r], spmem.at[li_v.at[r]], sem).wait()

        plsc.subcore_barrier()

        pltpu.sync_copy(
            spmem.at[pl.ds(sid * SL, SL)],
            wd_hbm.at[pl.ds(base + sid * SL, SL)],
        )


_densify = pl.kernel(
    _densify_body,
    mesh=_mesh,
    out_type=jax.ShapeDtypeStruct((N_W,), jnp.float32),
    scratch_types=_DENSIFY_SCRATCH,
)


def _mm_body(x_ref, w_ref, b_ref, o_ref):
    o_ref[...] = (
        jnp.dot(
            x_ref[...],
            w_ref[...].astype(jnp.bfloat16),
            preferred_element_type=jnp.float32,
        )
        + b_ref[...]
    )


def _matmul(x_bf, wd, bias2d):
    BN = 512
    return pl.pallas_call(
        _mm_body,
        grid=(OUT_F // BN,),
        in_specs=[
            pl.BlockSpec((B, IN_F), lambda j: (0, 0)),
            pl.BlockSpec((IN_F, BN), lambda j: (0, j)),
            pl.BlockSpec((1, BN), lambda j: (0, j)),
        ],
        out_specs=pl.BlockSpec((B, BN), lambda j: (0, j)),
        out_shape=jax.ShapeDtypeStruct((B, OUT_F), jnp.float32),
    )(x_bf, wd, bias2d)


def kernel(x, w_indices, w_values, bias):
    rows = w_indices[0].astype(jnp.int32)
    cols = w_indices[1].astype(jnp.int32)
    fidx = cols * OUT_F + rows
    pad = NNZP - NNZ
    fidx_p = jnp.concatenate([fidx, jnp.full((pad,), N_W, jnp.int32)])
    vals_p = jnp.concatenate([w_values.astype(jnp.float32),
                              jnp.zeros((pad,), jnp.float32)])
    wd_flat = _densify(fidx_p.reshape(NS, PW), vals_p.reshape(NS, G, 128))
    wd = wd_flat.reshape(IN_F, OUT_F)
    return _matmul(x.astype(jnp.bfloat16), wd, bias.reshape(1, OUT_F))


# speedup vs baseline: 1.1872x; 1.1872x over previous
import functools

import jax
import jax.numpy as jnp
from jax import lax
from jax.experimental import pallas as pl
from jax.experimental.pallas import tpu as pltpu
from jax.experimental.pallas import tpu_sc as plsc

IN_F = 4096
OUT_F = 4096
NNZ = 167772
B = 1024

N_W = IN_F * OUT_F
NC, NS = 2, 16
G = 84
PW = G * 128
NNZP = NS * PW
NCHUNK = 16
CH = N_W // NCHUNK
SL = CH // NS
ZB = 16384

_mesh = plsc.VectorSubcoreMesh(
    core_axis_name="c", subcore_axis_name="s", num_cores=NC, num_subcores=NS
)


G1 = G + 1

_DENSIFY_SCRATCH = [
    pltpu.VMEM((PW,), jnp.int32),
    pltpu.VMEM((G, 128), jnp.float32),
    pltpu.VMEM((G, 128), jnp.int32),
    pltpu.VMEM((ZB,), jnp.float32),
    pltpu.VMEM_SHARED((CH + 2048,), jnp.float32),
    pltpu.SemaphoreType.DMA,
    pltpu.SemaphoreType.DMA,
]


def _densify_body(fidx_hbm, val_hbm, wd_hbm, fidx_v, val_v, li_v, zbuf, spmem,
                  sem, zsem):
    cid = lax.axis_index("c")
    sid = lax.axis_index("s")

    @pl.loop(0, ZB // 16)
    def _zero(i):
        zbuf[pl.ds(i * 16, 16)] = jnp.zeros((16,), jnp.float32)

    pltpu.sync_copy(fidx_hbm.at[sid], fidx_v)
    pltpu.sync_copy(val_hbm.at[sid], val_v)

    for k in range(NCHUNK // NC):
        base = (k * NC + cid) * CH

        for z in range(SL // ZB):
            pltpu.async_copy(zbuf, spmem.at[pl.ds(sid * SL + z * ZB, ZB)], zsem)

        @pl.loop(0, G)
        def _locals(r):
            for cc in range(128 // 16):
                v = fidx_v[pl.ds(r * 128 + cc * 16, 16)]
                li = v - base
                ok = (li >= 0) & (li < CH)
                li_v[r, pl.ds(cc * 16, 16)] = jnp.where(ok, li, CH + (v & 2047))

        for z in range(SL // ZB):
            pltpu.make_async_copy(
                zbuf, spmem.at[pl.ds(sid * SL + z * ZB, ZB)], zsem
            ).wait()
        plsc.subcore_barrier()

        @pl.loop(0, G)
        def _scatter(r):
            pltpu.async_copy(val_v.at[r], spmem.at[li_v.at[r]], sem, add=True)

        @pl.loop(0, G)
        def _drain(r):
            pltpu.make_async_copy(val_v.at[r], spmem.at[li_v.at[r]], sem).wait()

        plsc.subcore_barrier()

        pltpu.sync_copy(
            spmem.at[pl.ds(sid * SL, SL)],
            wd_hbm.at[pl.ds(base + sid * SL, SL)],
        )


_densify = pl.kernel(
    _densify_body,
    mesh=_mesh,
    out_type=jax.ShapeDtypeStruct((N_W,), jnp.float32),
    scratch_types=_DENSIFY_SCRATCH,
)


PP = 2


def _mm_body(x_ref, w_ref, b_ref, o_ref):
    xb = x_ref[...]
    for p in range(PP):
        w = w_ref[pl.ds(p * IN_F, IN_F), :].astype(jnp.bfloat16)
        o_ref[:, pl.ds(p * 128, 128)] = (
            jnp.dot(xb, w, preferred_element_type=jnp.float32)
            + b_ref[:, pl.ds(p * 128, 128)]
        )


def _matmul(x_bf, wv, bias2d):
    return pl.pallas_call(
        _mm_body,
        grid=(OUT_F // (PP * 128),),
        in_specs=[
            pl.BlockSpec((B, IN_F), lambda j: (0, 0)),
            pl.BlockSpec((PP * IN_F, 128), lambda j: (j, 0)),
            pl.BlockSpec((1, PP * 128), lambda j: (0, j)),
        ],
        out_specs=pl.BlockSpec((B, PP * 128), lambda j: (0, j)),
        out_shape=jax.ShapeDtypeStruct((B, OUT_F), jnp.float32),
    )(x_bf, wv, bias2d)


def kernel(x, w_indices, w_values, bias):
    rows = w_indices[0].astype(jnp.int32)
    cols = w_indices[1].astype(jnp.int32)
    fidx = ((rows >> 7) * IN_F + cols) * 128 + (rows & 127)
    pad = NNZP - NNZ
    fidx_p = jnp.concatenate([fidx, jnp.full((pad,), N_W, jnp.int32)])
    vals_p = jnp.concatenate([w_values.astype(jnp.float32),
                              jnp.zeros((pad,), jnp.float32)])
    wd_flat = _densify(fidx_p.reshape(NS, PW), vals_p.reshape(NS, G, 128))
    wv = wd_flat.reshape(N_W // 128, 128)
    return _matmul(x.astype(jnp.bfloat16), wv, bias.reshape(1, OUT_F))


# speedup vs baseline: 1.3798x; 1.1622x over previous
import functools

import jax
import jax.numpy as jnp
from jax import lax
from jax.experimental import pallas as pl
from jax.experimental.pallas import tpu as pltpu
from jax.experimental.pallas import tpu_sc as plsc

IN_F = 4096
OUT_F = 4096
NNZ = 167772
B = 1024

N_W = IN_F * OUT_F
NC, NS = 2, 16
G = 84
PW = G * 128
NNZP = NS * PW
NCHUNK = 16
CH = N_W // NCHUNK
SL = CH // NS
ZB = 16384

_mesh = plsc.VectorSubcoreMesh(
    core_axis_name="c", subcore_axis_name="s", num_cores=NC, num_subcores=NS
)


G1 = G + 1

_DENSIFY_SCRATCH = [
    pltpu.VMEM((PW,), jnp.int32),
    pltpu.VMEM((G, 128), jnp.float32),
    pltpu.VMEM((G, 128), jnp.int32),
    pltpu.VMEM((ZB,), jnp.float32),
    pltpu.VMEM_SHARED((CH + 2048,), jnp.float32),
    pltpu.SemaphoreType.DMA,
    pltpu.SemaphoreType.DMA,
]


def _densify_body(fidx_hbm, val_hbm, wd_hbm, fidx_v, val_v, li_v, zbuf, spmem,
                  sem, zsem):
    cid = lax.axis_index("c")
    sid = lax.axis_index("s")

    @pl.loop(0, ZB // 16)
    def _zero(i):
        zbuf[pl.ds(i * 16, 16)] = jnp.zeros((16,), jnp.float32)

    pltpu.sync_copy(fidx_hbm.at[sid], fidx_v)
    pltpu.sync_copy(val_hbm.at[sid], val_v)

    for k in range(NCHUNK // NC):
        base = (k * NC + cid) * CH

        for z in range(SL // ZB):
            pltpu.async_copy(zbuf, spmem.at[pl.ds(sid * SL + z * ZB, ZB)], zsem)

        @pl.loop(0, G)
        def _locals(r):
            for cc in range(128 // 16):
                v = fidx_v[pl.ds(r * 128 + cc * 16, 16)]
                li = v - base
                ok = (li >= 0) & (li < CH)
                li_v[r, pl.ds(cc * 16, 16)] = jnp.where(ok, li, CH + (v & 2047))

        for z in range(SL // ZB):
            pltpu.make_async_copy(
                zbuf, spmem.at[pl.ds(sid * SL + z * ZB, ZB)], zsem
            ).wait()
        plsc.subcore_barrier()

        @pl.loop(0, G)
        def _scatter(r):
            pltpu.async_copy(val_v.at[r], spmem.at[li_v.at[r]], sem, add=True)

        @pl.loop(0, G)
        def _drain(r):
            pltpu.make_async_copy(val_v.at[r], spmem.at[li_v.at[r]], sem).wait()

        plsc.subcore_barrier()

        pltpu.sync_copy(
            spmem.at[pl.ds(sid * SL, SL)],
            wd_hbm.at[pl.ds(base + sid * SL, SL)],
        )


_densify = pl.kernel(
    _densify_body,
    mesh=_mesh,
    out_type=jax.ShapeDtypeStruct((N_W,), jnp.float32),
    scratch_types=_DENSIFY_SCRATCH,
)


PP = 4


def _mm_body(x_ref, w_ref, b_ref, o_ref):
    xb = x_ref[...]
    w = jnp.concatenate(
        [w_ref[pl.ds(p * IN_F, IN_F), :].astype(jnp.bfloat16)
         for p in range(PP)],
        axis=1,
    )
    o_ref[...] = (
        jnp.dot(xb, w, preferred_element_type=jnp.float32) + b_ref[...]
    )


def _matmul(x_bf, wv, bias2d):
    return pl.pallas_call(
        _mm_body,
        grid=(OUT_F // (PP * 128),),
        in_specs=[
            pl.BlockSpec((B, IN_F), lambda j: (0, 0)),
            pl.BlockSpec((PP * IN_F, 128), lambda j: (j, 0)),
            pl.BlockSpec((1, PP * 128), lambda j: (0, j)),
        ],
        out_specs=pl.BlockSpec((B, PP * 128), lambda j: (0, j)),
        out_shape=jax.ShapeDtypeStruct((B, OUT_F), jnp.float32),
    )(x_bf, wv, bias2d)


def kernel(x, w_indices, w_values, bias):
    rows = w_indices[0].astype(jnp.int32)
    cols = w_indices[1].astype(jnp.int32)
    fidx = ((rows >> 7) * IN_F + cols) * 128 + (rows & 127)
    pad = NNZP - NNZ
    fidx_p = jnp.concatenate([fidx, jnp.full((pad,), N_W, jnp.int32)])
    vals_p = jnp.concatenate([w_values.astype(jnp.float32),
                              jnp.zeros((pad,), jnp.float32)])
    wd_flat = _densify(fidx_p.reshape(NS, PW), vals_p.reshape(NS, G, 128))
    wv = wd_flat.reshape(N_W // 128, 128)
    return _matmul(x.astype(jnp.bfloat16), wv, bias.reshape(1, OUT_F))


# speedup vs baseline: 1.5076x; 1.0926x over previous
import functools

import jax
import jax.numpy as jnp
from jax import lax
from jax.experimental import pallas as pl
from jax.experimental.pallas import tpu as pltpu
from jax.experimental.pallas import tpu_sc as plsc

IN_F = 4096
OUT_F = 4096
NNZ = 167772
B = 1024

N_W = IN_F * OUT_F
NC, NS = 2, 16
G = 84
PW = G * 128
NNZP = NS * PW
NCHUNK = 12
CH = 1441792
N_PAD = NCHUNK * CH
SL = CH // NS
ZB = 4096

_mesh = plsc.VectorSubcoreMesh(
    core_axis_name="c", subcore_axis_name="s", num_cores=NC, num_subcores=NS
)


G1 = G + 1

_DENSIFY_SCRATCH = [
    pltpu.VMEM((PW,), jnp.int32),
    pltpu.VMEM((G, 128), jnp.float32),
    pltpu.VMEM((G, 128), jnp.int32),
    pltpu.VMEM((ZB,), jnp.float32),
    pltpu.VMEM_SHARED((CH + 2048,), jnp.float32),
    pltpu.SemaphoreType.DMA,
    pltpu.SemaphoreType.DMA,
]


def _densify_body(fidx_hbm, val_hbm, wd_hbm, fidx_v, val_v, li_v, zbuf, spmem,
                  sem, zsem):
    cid = lax.axis_index("c")
    sid = lax.axis_index("s")

    @pl.loop(0, ZB // 16)
    def _zero(i):
        zbuf[pl.ds(i * 16, 16)] = jnp.zeros((16,), jnp.float32)

    pltpu.sync_copy(fidx_hbm.at[sid], fidx_v)
    pltpu.sync_copy(val_hbm.at[sid], val_v)

    for k in range(NCHUNK // NC):
        base = (k * NC + cid) * CH

        for z in range(SL // ZB):
            pltpu.async_copy(zbuf, spmem.at[pl.ds(sid * SL + z * ZB, ZB)], zsem)

        @pl.loop(0, G)
        def _locals(r):
            for cc in range(128 // 16):
                v = fidx_v[pl.ds(r * 128 + cc * 16, 16)]
                li = v - base
                ok = (li >= 0) & (li < CH)
                li_v[r, pl.ds(cc * 16, 16)] = jnp.where(ok, li, CH + (v & 2047))

        for z in range(SL // ZB):
            pltpu.make_async_copy(
                zbuf, spmem.at[pl.ds(sid * SL + z * ZB, ZB)], zsem
            ).wait()
        plsc.subcore_barrier()

        @pl.loop(0, G)
        def _scatter(r):
            pltpu.async_copy(val_v.at[r], spmem.at[li_v.at[r]], sem, add=True)

        @pl.loop(0, G)
        def _drain(r):
            pltpu.make_async_copy(val_v.at[r], spmem.at[li_v.at[r]], sem).wait()

        plsc.subcore_barrier()

        pltpu.sync_copy(
            spmem.at[pl.ds(sid * SL, SL)],
            wd_hbm.at[pl.ds(base + sid * SL, SL)],
        )


_densify = pl.kernel(
    _densify_body,
    mesh=_mesh,
    out_type=jax.ShapeDtypeStruct((N_PAD,), jnp.float32),
    scratch_types=_DENSIFY_SCRATCH,
)


PP = 4


def _mm_body(x_ref, w_ref, b_ref, o_ref):
    xb = x_ref[...]
    w = jnp.concatenate(
        [w_ref[pl.ds(p * IN_F, IN_F), :].astype(jnp.bfloat16)
         for p in range(PP)],
        axis=1,
    )
    o_ref[...] = (
        jnp.dot(xb, w, preferred_element_type=jnp.float32) + b_ref[...]
    )


def _matmul(x_bf, wv, bias2d):
    return pl.pallas_call(
        _mm_body,
        grid=(OUT_F // (PP * 128),),
        in_specs=[
            pl.BlockSpec((B, IN_F), lambda j: (0, 0)),
            pl.BlockSpec((PP * IN_F, 128), lambda j: (j, 0)),
            pl.BlockSpec((1, PP * 128), lambda j: (0, j)),
        ],
        out_specs=pl.BlockSpec((B, PP * 128), lambda j: (0, j)),
        out_shape=jax.ShapeDtypeStruct((B, OUT_F), jnp.float32),
    )(x_bf, wv, bias2d)


def kernel(x, w_indices, w_values, bias):
    rows = w_indices[0].astype(jnp.int32)
    cols = w_indices[1].astype(jnp.int32)
    fidx = ((rows >> 7) * IN_F + cols) * 128 + (rows & 127)
    pad = NNZP - NNZ
    fidx_p = jnp.concatenate([fidx, jnp.full((pad,), N_PAD, jnp.int32)])
    vals_p = jnp.concatenate([w_values.astype(jnp.float32),
                              jnp.zeros((pad,), jnp.float32)])
    wd_flat = _densify(fidx_p.reshape(NS, PW), vals_p.reshape(NS, G, 128))
    wv = wd_flat.reshape(N_PAD // 128, 128)
    return _matmul(x.astype(jnp.bfloat16), wv, bias.reshape(1, OUT_F))


# speedup vs baseline: 1.7557x; 1.1646x over previous
import functools

import jax
import jax.numpy as jnp
from jax import lax
from jax.experimental import pallas as pl
from jax.experimental.pallas import tpu as pltpu
from jax.experimental.pallas import tpu_sc as plsc

IN_F = 4096
OUT_F = 4096
NNZ = 167772
B = 1024

N_W = IN_F * OUT_F
NC, NS = 2, 16
G = 82
PW = G * 128
NNZP = NS * PW
NCHUNK = 10
CH = 1703936
N_PAD = NCHUNK * CH
SL = CH // NS
ZB = 1024

_mesh = plsc.VectorSubcoreMesh(
    core_axis_name="c", subcore_axis_name="s", num_cores=NC, num_subcores=NS
)


G1 = G + 1

_DENSIFY_SCRATCH = [
    pltpu.VMEM((G, 128), jnp.int32),
    pltpu.VMEM((G, 128), jnp.float32),
    pltpu.VMEM((ZB,), jnp.float32),
    pltpu.VMEM_SHARED((CH + 2048,), jnp.float32),
    pltpu.SemaphoreType.DMA,
    pltpu.SemaphoreType.DMA,
]


def _densify_body(fidx_hbm, val_hbm, wd_hbm, li_v, val_v, zbuf, spmem,
                  sem, zsem):
    cid = lax.axis_index("c")
    sid = lax.axis_index("s")

    @pl.loop(0, ZB // 16)
    def _zero(i):
        zbuf[pl.ds(i * 16, 16)] = jnp.zeros((16,), jnp.float32)

    pltpu.sync_copy(val_hbm.at[sid], val_v)

    for k in range(NCHUNK // NC):
        base = (k * NC + cid) * CH

        for z in range(SL // ZB):
            pltpu.async_copy(zbuf, spmem.at[pl.ds(sid * SL + z * ZB, ZB)], zsem)

        pltpu.sync_copy(fidx_hbm.at[sid], li_v)

        @pl.loop(0, G)
        def _locals(r):
            for cc in range(128 // 16):
                v = li_v[r, pl.ds(cc * 16, 16)]
                li = v - base
                ok = (li >= 0) & (li < CH)
                li_v[r, pl.ds(cc * 16, 16)] = jnp.where(ok, li, CH + (v & 2047))

        for z in range(SL // ZB):
            pltpu.make_async_copy(
                zbuf, spmem.at[pl.ds(sid * SL + z * ZB, ZB)], zsem
            ).wait()
        plsc.subcore_barrier()

        @pl.loop(0, G)
        def _scatter(r):
            pltpu.async_copy(val_v.at[r], spmem.at[li_v.at[r]], sem, add=True)

        @pl.loop(0, G)
        def _drain(r):
            pltpu.make_async_copy(val_v.at[r], spmem.at[li_v.at[r]], sem).wait()

        plsc.subcore_barrier()

        pltpu.sync_copy(
            spmem.at[pl.ds(sid * SL, SL)],
            wd_hbm.at[pl.ds(base + sid * SL, SL)],
        )


_densify = pl.kernel(
    _densify_body,
    mesh=_mesh,
    out_type=jax.ShapeDtypeStruct((N_PAD,), jnp.float32),
    scratch_types=_DENSIFY_SCRATCH,
)


PP = 4


def _mm_body(x_ref, w_ref, b_ref, o_ref):
    xb = x_ref[...]
    w = jnp.concatenate(
        [w_ref[pl.ds(p * IN_F, IN_F), :].astype(jnp.bfloat16)
         for p in range(PP)],
        axis=1,
    )
    o_ref[...] = (
        jnp.dot(xb, w, preferred_element_type=jnp.float32) + b_ref[...]
    )


def _matmul(x_bf, wv, bias2d):
    return pl.pallas_call(
        _mm_body,
        grid=(OUT_F // (PP * 128),),
        in_specs=[
            pl.BlockSpec((B, IN_F), lambda j: (0, 0)),
            pl.BlockSpec((PP * IN_F, 128), lambda j: (j, 0)),
            pl.BlockSpec((1, PP * 128), lambda j: (0, j)),
        ],
        out_specs=pl.BlockSpec((B, PP * 128), lambda j: (0, j)),
        out_shape=jax.ShapeDtypeStruct((B, OUT_F), jnp.float32),
    )(x_bf, wv, bias2d)


def kernel(x, w_indices, w_values, bias):
    rows = w_indices[0].astype(jnp.int32)
    cols = w_indices[1].astype(jnp.int32)
    fidx = ((rows >> 7) * IN_F + cols) * 128 + (rows & 127)
    pad = NNZP - NNZ
    fidx_p = jnp.concatenate([fidx, jnp.full((pad,), N_PAD, jnp.int32)])
    vals_p = jnp.concatenate([w_values.astype(jnp.float32),
                              jnp.zeros((pad,), jnp.float32)])
    wd_flat = _densify(fidx_p.reshape(NS, G, 128), vals_p.reshape(NS, G, 128))
    wv = wd_flat.reshape(N_PAD // 128, 128)
    return _matmul(x.astype(jnp.bfloat16), wv, bias.reshape(1, OUT_F))


# speedup vs baseline: 1.7569x; 1.0007x over previous
import jax
import jax.numpy as jnp
from jax import lax
from jax.experimental import pallas as pl
from jax.experimental.pallas import tpu as pltpu
from jax.experimental.pallas import tpu_sc as plsc

IN_F = 4096
OUT_F = 4096
NNZ = 167772
B = 1024

N_W = IN_F * OUT_F
NC, NS = 2, 16
G = 82
PW = G * 128
NNZP = NS * PW
NCHUNK = 10
CH = 1703936
N_PAD = NCHUNK * CH
SL = CH // NS
ZB = 1024

_mesh = plsc.VectorSubcoreMesh(
    core_axis_name="c", subcore_axis_name="s", num_cores=NC, num_subcores=NS
)


_DENSIFY_SCRATCH = [
    pltpu.VMEM((G, 128), jnp.int32),
    pltpu.VMEM((G, 128), jnp.float32),
    pltpu.VMEM((ZB,), jnp.float32),
    pltpu.VMEM_SHARED((CH + 2048,), jnp.float32),
    pltpu.SemaphoreType.DMA,
    pltpu.SemaphoreType.DMA,
]


def _densify_body(fidx_hbm, val_hbm, wd_hbm, li_v, val_v, zbuf, spmem,
                  sem, zsem):
    cid = lax.axis_index("c")
    sid = lax.axis_index("s")

    @pl.loop(0, ZB // 16)
    def _zero(i):
        zbuf[pl.ds(i * 16, 16)] = jnp.zeros((16,), jnp.float32)

    pltpu.sync_copy(val_hbm.at[sid], val_v)

    for k in range(NCHUNK // NC):
        base = (k * NC + cid) * CH

        for z in range(SL // ZB):
            pltpu.async_copy(zbuf, spmem.at[pl.ds(sid * SL + z * ZB, ZB)], zsem)

        pltpu.sync_copy(fidx_hbm.at[sid], li_v)

        @pl.loop(0, G)
        def _locals(r):
            for cc in range(128 // 16):
                v = li_v[r, pl.ds(cc * 16, 16)]
                li = v - base
                ok = (li >= 0) & (li < CH)
                li_v[r, pl.ds(cc * 16, 16)] = jnp.where(ok, li, CH + (v & 2047))

        for z in range(SL // ZB):
            pltpu.make_async_copy(
                zbuf, spmem.at[pl.ds(sid * SL + z * ZB, ZB)], zsem
            ).wait()
        plsc.subcore_barrier()

        @pl.loop(0, G)
        def _scatter(r):
            pltpu.async_copy(val_v.at[r], spmem.at[li_v.at[r]], sem, add=True)

        @pl.loop(0, G)
        def _drain(r):
            pltpu.make_async_copy(val_v.at[r], spmem.at[li_v.at[r]], sem).wait()

        plsc.subcore_barrier()

        pltpu.sync_copy(
            spmem.at[pl.ds(sid * SL, SL)],
            wd_hbm.at[pl.ds(base + sid * SL, SL)],
        )


_densify = pl.kernel(
    _densify_body,
    mesh=_mesh,
    out_type=jax.ShapeDtypeStruct((N_PAD,), jnp.float32),
    scratch_types=_DENSIFY_SCRATCH,
)


PP = 4


def _mm_body(x_ref, w_ref, b_ref, o_ref):
    xb = x_ref[...]
    w = jnp.concatenate(
        [w_ref[pl.ds(p * IN_F, IN_F), :].astype(jnp.bfloat16)
         for p in range(PP)],
        axis=1,
    )
    o_ref[...] = (
        jnp.dot(xb, w, preferred_element_type=jnp.float32) + b_ref[...]
    )


def _matmul(x_bf, wv, bias2d):
    return pl.pallas_call(
        _mm_body,
        grid=(OUT_F // (PP * 128),),
        in_specs=[
            pl.BlockSpec((B, IN_F), lambda j: (0, 0)),
            pl.BlockSpec((PP * IN_F, 128), lambda j: (j, 0)),
            pl.BlockSpec((1, PP * 128), lambda j: (0, j)),
        ],
        out_specs=pl.BlockSpec((B, PP * 128), lambda j: (0, j)),
        out_shape=jax.ShapeDtypeStruct((B, OUT_F), jnp.float32),
    )(x_bf, wv, bias2d)


def kernel(x, w_indices, w_values, bias):
    rows = w_indices[0].astype(jnp.int32)
    cols = w_indices[1].astype(jnp.int32)
    fidx = ((rows >> 7) * IN_F + cols) * 128 + (rows & 127)
    pad = NNZP - NNZ
    fidx_p = jnp.concatenate([fidx, jnp.full((pad,), N_PAD, jnp.int32)])
    vals_p = jnp.concatenate([w_values.astype(jnp.float32),
                              jnp.zeros((pad,), jnp.float32)])
    wd_flat = _densify(fidx_p.reshape(NS, G, 128), vals_p.reshape(NS, G, 128))
    wv = wd_flat.reshape(N_PAD // 128, 128)
    return _matmul(x.astype(jnp.bfloat16), wv, bias.reshape(1, OUT_F))
